# baseline (speedup 1.0000x reference)
import functools

import jax
import jax.numpy as jnp
from jax import lax
from jax.experimental import pallas as pl
from jax.experimental.pallas import tpu as pltpu
from jax.experimental.pallas import tpu_sc as plsc

ROWS, COLS = 16384, 136
NC, NS, L = 2, 16, 16
NW = NC * NS
RPW = ROWS // NW
CHUNK = 128
NCHUNK = RPW // CHUNK
NGRP = CHUNK // L
SEGS = [(0, 8)] + [(8 + 16 * k, 16) for k in range(8)]


def _tree(vals, op):
    while len(vals) > 1:
        vals = [op(vals[i], vals[i + 1]) if i + 1 < len(vals) else vals[i]
                for i in range(0, len(vals), 2)]
    return vals[0]


def _root_group(buf, g):
    row_idx = g * L + lax.iota(jnp.int32, L)
    cidx = [jnp.full((L,), j, jnp.int32) for j in range(8)]
    cols = [plsc.load_gather(buf, [row_idx, cidx[j]]) for j in range(8)]
    m = _tree(cols, jnp.maximum)
    es = [jnp.exp(c - m) for c in cols]
    s = _tree(es, lax.add)
    inv = 1.0 / s
    for j in range(8):
        plsc.store_scatter(buf, [row_idx, cidx[j]], es[j] * inv)


def _child_row(buf, r):
    iota = lax.iota(jnp.int32, L)
    row = jnp.full((L,), r, jnp.int32)
    for k in range(8):
        cidx = 8 + 16 * k + iota
        col = plsc.load_gather(buf, [row, cidx])
        m = jnp.max(col)
        e = jnp.exp(col - m)
        s = jnp.broadcast_to(jnp.sum(e), (L,))
        plsc.store_scatter(buf, [row, cidx], e * (1.0 / s))


_MESH = plsc.VectorSubcoreMesh(core_axis_name="c", subcore_axis_name="s")


@functools.partial(
    pl.kernel,
    mesh=_MESH,
    out_type=jax.ShapeDtypeStruct((ROWS, COLS), jnp.float32),
    scratch_types=[
        pltpu.VMEM((CHUNK, COLS), jnp.float32),
        pltpu.VMEM((CHUNK, COLS), jnp.float32),
        pltpu.SemaphoreType.DMA,
        pltpu.SemaphoreType.DMA,
        pltpu.SemaphoreType.DMA,
        pltpu.SemaphoreType.DMA,
    ],
    compiler_params=pltpu.CompilerParams(needs_layout_passes=False),
)
def _hsoftmax(x_hbm, out_hbm, buf0, buf1, si0, si1, so0, so1):
    wid = lax.axis_index("s") * NC + lax.axis_index("c")
    base = wid * RPW
    bufs, isems, osems = (buf0, buf1), (si0, si1), (so0, so1)

    def in_slice(ci):
        return x_hbm.at[pl.ds(base + ci * CHUNK, CHUNK)]

    def out_slice(ci):
        return out_hbm.at[pl.ds(base + ci * CHUNK, CHUNK)]

    def bview(i):
        return bufs[i]

    pltpu.async_copy(in_slice(0), bview(0), isems[0])
    for ci in range(NCHUNK):
        b = ci % 2
        nb = (ci + 1) % 2
        pltpu.make_async_copy(in_slice(ci), bview(b), isems[b]).wait()
        if ci + 1 < NCHUNK:
            if ci >= 1:
                pltpu.make_async_copy(bview(nb), out_slice(ci - 1),
                                      osems[nb]).wait()
            pltpu.async_copy(in_slice(ci + 1), bview(nb), isems[nb])

        @plsc.parallel_loop(0, NGRP, unroll=2)
        def root_body(g, _b=b):
            _root_group(bufs[_b], g)

        @plsc.parallel_loop(0, CHUNK, unroll=8)
        def child_body(r, _b=b):
            _child_row(bufs[_b], r)
        pltpu.async_copy(bview(b), out_slice(ci), osems[b])
    pltpu.make_async_copy(bview((NCHUNK - 2) % 2), out_slice(NCHUNK - 2),
                          osems[(NCHUNK - 2) % 2]).wait()
    pltpu.make_async_copy(bview((NCHUNK - 1) % 2), out_slice(NCHUNK - 1),
                          osems[(NCHUNK - 1) % 2]).wait()


def kernel(x):
    return _hsoftmax(x)


# speedup vs baseline: 2.6502x; 2.6502x over previous
import functools

import jax
import jax.numpy as jnp
from jax import lax
from jax.experimental import pallas as pl
from jax.experimental.pallas import tpu as pltpu
from jax.experimental.pallas import tpu_sc as plsc

ROWS, COLS = 16384, 136
NC, NS, L = 2, 16, 16
NW = NC * NS
RPW = ROWS // NW
CHUNK = 128
NCHUNK = RPW // CHUNK
NGRP = CHUNK // L
SEGS = [(0, 8)] + [(8 + 16 * k, 16) for k in range(8)]


def _tree(vals, op):
    while len(vals) > 1:
        vals = [op(vals[i], vals[i + 1]) if i + 1 < len(vals) else vals[i]
                for i in range(0, len(vals), 2)]
    return vals[0]


def _root_group(buf, g):
    row_idx = g * L + lax.iota(jnp.int32, L)
    cidx = [jnp.full((L,), j, jnp.int32) for j in range(8)]
    cols = [plsc.load_gather(buf, [row_idx, cidx[j]]) for j in range(8)]
    m = _tree(cols, jnp.maximum)
    es = [jnp.exp(c - m) for c in cols]
    s = _tree(es, lax.add)
    inv = 1.0 / s
    for j in range(8):
        plsc.store_scatter(buf, [row_idx, cidx[j]], es[j] * inv)


def _child_row(buf, r):
    for k in range(8):
        c0 = 8 + 16 * k
        col = buf[r, pl.ds(c0, L)]
        m = jnp.max(col)
        e = jnp.exp(col - m)
        s = jnp.broadcast_to(jnp.sum(e), (L,))
        buf[r, pl.ds(c0, L)] = e * (1.0 / s)


_MESH = plsc.VectorSubcoreMesh(core_axis_name="c", subcore_axis_name="s")


@functools.partial(
    pl.kernel,
    mesh=_MESH,
    out_type=jax.ShapeDtypeStruct((ROWS, COLS), jnp.float32),
    scratch_types=[
        pltpu.VMEM((CHUNK, COLS), jnp.float32),
        pltpu.VMEM((CHUNK, COLS), jnp.float32),
        pltpu.SemaphoreType.DMA,
        pltpu.SemaphoreType.DMA,
        pltpu.SemaphoreType.DMA,
        pltpu.SemaphoreType.DMA,
    ],
    compiler_params=pltpu.CompilerParams(needs_layout_passes=False),
)
def _hsoftmax(x_hbm, out_hbm, buf0, buf1, si0, si1, so0, so1):
    wid = lax.axis_index("s") * NC + lax.axis_index("c")
    base = wid * RPW
    bufs, isems, osems = (buf0, buf1), (si0, si1), (so0, so1)

    def in_slice(ci):
        return x_hbm.at[pl.ds(base + ci * CHUNK, CHUNK)]

    def out_slice(ci):
        return out_hbm.at[pl.ds(base + ci * CHUNK, CHUNK)]

    def bview(i):
        return bufs[i]

    pltpu.async_copy(in_slice(0), bview(0), isems[0])
    for ci in range(NCHUNK):
        b = ci % 2
        nb = (ci + 1) % 2
        pltpu.make_async_copy(in_slice(ci), bview(b), isems[b]).wait()
        if ci + 1 < NCHUNK:
            if ci >= 1:
                pltpu.make_async_copy(bview(nb), out_slice(ci - 1),
                                      osems[nb]).wait()
            pltpu.async_copy(in_slice(ci + 1), bview(nb), isems[nb])

        @plsc.parallel_loop(0, NGRP, unroll=1)
        def root_body(g, _b=b):
            _root_group(bufs[_b], g)

        @plsc.parallel_loop(0, CHUNK, unroll=4)
        def child_body(r, _b=b):
            _child_row(bufs[_b], r)
        pltpu.async_copy(bview(b), out_slice(ci), osems[b])
    pltpu.make_async_copy(bview((NCHUNK - 2) % 2), out_slice(NCHUNK - 2),
                          osems[(NCHUNK - 2) % 2]).wait()
    pltpu.make_async_copy(bview((NCHUNK - 1) % 2), out_slice(NCHUNK - 1),
                          osems[(NCHUNK - 1) % 2]).wait()


def kernel(x):
    return _hsoftmax(x)
